# baseline (speedup 1.0000x reference)
import jax
import jax.numpy as jnp
from jax.experimental import pallas as pl

N = 10000
B = 50
NPER = 200
K = 8
SLOPE = 0.01


def _bf(t):
    return t.astype(jnp.bfloat16)


def _mm(a, b):
    return jax.lax.dot_general(a, b, (((1,), (0,)), ((), ())),
                               preferred_element_type=jnp.float32)


def _leaky(t):
    return jnp.where(t >= 0, t, SLOPE * t)


def _split3(x):
    h1 = _bf(x)
    r = x - h1.astype(jnp.float32)
    h2 = _bf(r)
    h3 = _bf(r - h2.astype(jnp.float32))
    return h1, h2, h3


def _body(x_ref, np_ref,
          A1, B1, c1b1, W21, c1b2,
          A2, B2, c2b1, W22, c2b2,
          A3, B3, c3b1, W23, c3b2,
          A4, B4, c4b1, W24, c4b2,
          sx, sa, sb, sc, sd, nb1,
          n2T, nb2,
          tmx, tmn, tsm, tmean, textra, nb3,
          n4T, nb4,
          out_ref):
    X = x_ref[...]
    ci = jax.lax.broadcasted_iota(jnp.int32, (NPER, NPER), 1)
    ri = jax.lax.broadcasted_iota(jnp.int32, (NPER, NPER), 0)
    cie = jax.lax.broadcasted_iota(jnp.int32, (NPER * K, NPER), 1)
    eyeb = (ri == ci).astype(jnp.bfloat16)
    eye_big = (ri == ci).astype(jnp.float32) * 1e10

    def knn(P):
        p1, p2, p3 = _split3(P)
        Pc = jnp.concatenate([p1, p2, p3], axis=1)
        PT = jax.lax.dot_general(Pc, eyeb, (((0,), (0,)), ((), ())),
                                 preferred_element_type=jnp.float32)
        d2 = None
        for c in range(3):
            col = P[:, c:c + 1]
            row = (PT[c:c + 1] + PT[c + 3:c + 4]) + PT[c + 6:c + 7]
            dc = col - row
            sq = dc * dc
            d2 = sq if d2 is None else d2 + sq
        d2 = d2 + eye_big
        ohs = []
        work = d2
        for _ in range(K):
            m = jnp.min(work, axis=1, keepdims=True)
            cand = jnp.where(work == m, ci, jnp.int32(2147483647))
            amin = jnp.min(cand, axis=1, keepdims=True)
            oh = ci == amin
            ohs.append(oh)
            work = jnp.where(oh, jnp.float32(jnp.inf), work)
        return ohs

    def conv(Xin, cols, WaT, WbT, b1r, W2T, b2r, hom=False):
        chunks = _split3(Xin)
        Un = _mm(chunks[0], WaT[...])
        ohbs = [oh.astype(jnp.bfloat16) for oh in cols]
        g1s = [_mm(ohb, chunks[0]) for ohb in ohbs]
        g2s = [_mm(ohb, chunks[1]) for ohb in ohbs]
        g3s = [_mm(ohb, chunks[2]) for ohb in ohbs]
        Vks = [(g1 + g2) + g3 for g1, g2, g3 in zip(g1s, g2s, g3s)]
        Dks = [_bf(Vk - Xin) for Vk in Vks]
        P1s = [(Un + _mm(Dk, WbT[...])) + b1r[...] for Dk in Dks]
        E1s = [_bf(_leaky(P1)) for P1 in P1s]
        E2s = [_leaky(_mm(E1, W2T[...]) + b2r[...]) for E1 in E1s]
        acc = None
        for E2 in E2s:
            acc = E2 if acc is None else acc + E2
        if not hom:
            return acc
        hsum = None
        for Vk in Vks:
            same = (Vk[:, 0:4] == Xin[:, 0:4]).astype(jnp.float32)
            s = jnp.sum(same, axis=0, keepdims=True)
            hsum = s if hsum is None else hsum + s
        return acc, hsum * (1.0 / (NPER * K))

    src1 = knn(X[:, 0:3])
    a, h4 = conv(X, src1, A1, B1, c1b1, W21, c1b2, hom=True)
    src2 = knn(a[:, 0:3])
    b = conv(a, src2, A2, B2, c2b1, W22, c2b2)
    src3 = knn(b[:, 0:3])
    c = conv(b, src3, A3, B3, c3b1, W23, c3b2)
    src4 = knn(c[:, 0:3])
    d = conv(c, src4, A4, B4, c4b1, W24, c4b2)

    H = _leaky((((( _mm(_bf(X), sx[...]) + _mm(_bf(a), sa[...]))
                 + _mm(_bf(b), sb[...])) + _mm(_bf(c), sc[...]))
                + _mm(_bf(d), sd[...])) + nb1[...])
    H2 = _mm(_bf(H), n2T[...]) + nb2[...]

    mx = jnp.max(H2, axis=0, keepdims=True)
    mn = jnp.min(H2, axis=0, keepdims=True)
    sm = jnp.sum(H2, axis=0, keepdims=True)
    mean = sm * (1.0 / NPER)

    npul = np_ref[0]
    ev = jnp.concatenate([h4[:, 3:4], h4[:, 0:1], h4[:, 1:2], h4[:, 2:3],
                          npul], axis=1)

    G = ((((_mm(_bf(_leaky(mx)), tmx[...]) + _mm(_bf(_leaky(mn)), tmn[...]))
           + _mm(_bf(_leaky(sm)), tsm[...]))
          + _mm(_bf(_leaky(mean)), tmean[...]))
         + _mm(_bf(_leaky(ev)), textra[...]))
    G = _leaky(G + nb3[...])
    O = _mm(_bf(G), n4T[...]) + nb4[...]
    out_ref[0] = jnp.concatenate([jnp.tanh(O[:, 0:2]), O[:, 2:3]], axis=1)


def kernel(x, batch, n_pulses,
           c1_W1, c1_b1, c1_W2, c1_b2,
           c2_W1, c2_b1, c2_W2, c2_b2,
           c3_W1, c3_b1, c3_W2, c3_b2,
           c4_W1, c4_b1, c4_W2, c4_b2,
           nn1_W, nn1_b, nn2_W, nn2_b,
           nn3_W, nn3_b, nn4_W, nn4_b):
    bf = lambda t: t.astype(jnp.bfloat16)
    weights = []
    for (W1, b1, W2, b2) in ((c1_W1, c1_b1, c1_W2, c1_b2),
                             (c2_W1, c2_b1, c2_W2, c2_b2),
                             (c3_W1, c3_b1, c3_W2, c3_b2),
                             (c4_W1, c4_b1, c4_W2, c4_b2)):
        din = W1.shape[1] // 2
        weights += [bf(W1[:, :din].T), bf(W1[:, din:].T),
                    b1[None, :], bf(W2.T), b2[None, :]]

    s = bf(nn1_W.T)
    weights += [s[0:7], s[7:263], s[263:519], s[519:775], s[775:1031],
                nn1_b[None, :]]
    weights += [bf(nn2_W.T), nn2_b[None, :]]
    t = bf(nn3_W.T)
    weights += [t[0:256], t[256:512], t[512:768], t[768:1024], t[1024:1029],
                nn3_b[None, :]]
    weights += [bf(nn4_W.T), nn4_b[None, :]]

    np2 = n_pulses.reshape(B, 1, 1)

    wspecs = [pl.BlockSpec(w.shape, lambda g: (0, 0)) for w in weights]
    out = pl.pallas_call(
        _body,
        grid=(B,),
        in_specs=[pl.BlockSpec((NPER, 7), lambda g: (g, 0)),
                  pl.BlockSpec((1, 1, 1), lambda g: (g, 0, 0))] + wspecs,
        out_specs=pl.BlockSpec((1, 1, 3), lambda g: (g, 0, 0)),
        out_shape=jax.ShapeDtypeStruct((B, 1, 3), jnp.float32),
    )(x, np2, *weights)
    return out.reshape(B, 3)


# speedup vs baseline: 1.1937x; 1.1937x over previous
import jax
import jax.numpy as jnp
from jax.experimental import pallas as pl

N = 10000
B = 50
NPER = 200
K = 8
SLOPE = 0.01


def _bf(t):
    return t.astype(jnp.bfloat16)


def _mm(a, b):
    return jax.lax.dot_general(a, b, (((1,), (0,)), ((), ())),
                               preferred_element_type=jnp.float32)


def _leaky(t):
    return jnp.where(t >= 0, t, SLOPE * t)


def _split3(x):
    h1 = _bf(x)
    r = x - h1.astype(jnp.float32)
    h2 = _bf(r)
    h3 = _bf(r - h2.astype(jnp.float32))
    return h1, h2, h3


def _body(x_ref, np_ref,
          A1, B1, c1b1, W21, c1b2,
          A2, B2, c2b1, W22, c2b2,
          A3, B3, c3b1, W23, c3b2,
          A4, B4, c4b1, W24, c4b2,
          sx, sa, sb, sc, sd, nb1,
          n2T, nb2,
          tmx, tmn, tsm, tmean, textra, nb3,
          n4T, nb4,
          out_ref):
    X = x_ref[...]
    ci = jax.lax.broadcasted_iota(jnp.int32, (NPER, NPER), 1)
    ri = jax.lax.broadcasted_iota(jnp.int32, (NPER, NPER), 0)
    cif = ci.astype(jnp.float32)
    eyeb = (ri == ci).astype(jnp.bfloat16)
    eye_big = (ri == ci).astype(jnp.float32) * 1e10

    def knn(P):
        p1, p2, p3 = _split3(P)
        Pc = jnp.concatenate([p1, p2, p3], axis=1)
        PT = jax.lax.dot_general(Pc, eyeb, (((0,), (0,)), ((), ())),
                                 preferred_element_type=jnp.float32)
        d2 = None
        for c in range(3):
            col = P[:, c:c + 1]
            row = (PT[c:c + 1] + PT[c + 3:c + 4]) + PT[c + 6:c + 7]
            dc = col - row
            sq = dc * dc
            d2 = sq if d2 is None else d2 + sq
        d2 = d2 + eye_big
        ohs = []
        work = d2
        for _ in range(K):
            m = jnp.min(work, axis=1, keepdims=True)
            cand = jnp.where(work == m, cif, jnp.float32(1e9))
            amin = jnp.min(cand, axis=1, keepdims=True)
            oh = cif == amin
            ohs.append(oh)
            work = jnp.where(oh, jnp.float32(jnp.inf), work)
        return ohs

    def conv(Xin, cols, WaT, WbT, b1r, W2T, b2r, hom=False):
        chunks = _split3(Xin)
        Un = _mm(chunks[0], WaT[...])
        ohbs = [oh.astype(jnp.bfloat16) for oh in cols]
        g1s = [_mm(ohb, chunks[0]) for ohb in ohbs]
        g2s = [_mm(ohb, chunks[1]) for ohb in ohbs]
        g3s = [_mm(ohb, chunks[2]) for ohb in ohbs]
        Vks = [(g1 + g2) + g3 for g1, g2, g3 in zip(g1s, g2s, g3s)]
        Dks = [_bf(Vk - Xin) for Vk in Vks]
        P1s = [(Un + _mm(Dk, WbT[...])) + b1r[...] for Dk in Dks]
        E1s = [_bf(_leaky(P1)) for P1 in P1s]
        E2s = [_leaky(_mm(E1, W2T[...]) + b2r[...]) for E1 in E1s]
        acc = None
        for E2 in E2s:
            acc = E2 if acc is None else acc + E2
        if not hom:
            return acc
        hsum = None
        for Vk in Vks:
            same = (Vk[:, 0:4] == Xin[:, 0:4]).astype(jnp.float32)
            s = jnp.sum(same, axis=0, keepdims=True)
            hsum = s if hsum is None else hsum + s
        return acc, hsum * (1.0 / (NPER * K))

    src1 = knn(X[:, 0:3])
    a, h4 = conv(X, src1, A1, B1, c1b1, W21, c1b2, hom=True)
    src2 = knn(a[:, 0:3])
    b = conv(a, src2, A2, B2, c2b1, W22, c2b2)
    src3 = knn(b[:, 0:3])
    c = conv(b, src3, A3, B3, c3b1, W23, c3b2)
    src4 = knn(c[:, 0:3])
    d = conv(c, src4, A4, B4, c4b1, W24, c4b2)

    H = _leaky((((( _mm(_bf(X), sx[...]) + _mm(_bf(a), sa[...]))
                 + _mm(_bf(b), sb[...])) + _mm(_bf(c), sc[...]))
                + _mm(_bf(d), sd[...])) + nb1[...])
    H2 = _mm(_bf(H), n2T[...]) + nb2[...]

    mx = jnp.max(H2, axis=0, keepdims=True)
    mn = jnp.min(H2, axis=0, keepdims=True)
    sm = jnp.sum(H2, axis=0, keepdims=True)
    mean = sm * (1.0 / NPER)

    npul = np_ref[0]
    ev = jnp.concatenate([h4[:, 3:4], h4[:, 0:1], h4[:, 1:2], h4[:, 2:3],
                          npul], axis=1)

    G = ((((_mm(_bf(_leaky(mx)), tmx[...]) + _mm(_bf(_leaky(mn)), tmn[...]))
           + _mm(_bf(_leaky(sm)), tsm[...]))
          + _mm(_bf(_leaky(mean)), tmean[...]))
         + _mm(_bf(_leaky(ev)), textra[...]))
    G = _leaky(G + nb3[...])
    O = _mm(_bf(G), n4T[...]) + nb4[...]
    out_ref[0] = jnp.concatenate([jnp.tanh(O[:, 0:2]), O[:, 2:3]], axis=1)


def kernel(x, batch, n_pulses,
           c1_W1, c1_b1, c1_W2, c1_b2,
           c2_W1, c2_b1, c2_W2, c2_b2,
           c3_W1, c3_b1, c3_W2, c3_b2,
           c4_W1, c4_b1, c4_W2, c4_b2,
           nn1_W, nn1_b, nn2_W, nn2_b,
           nn3_W, nn3_b, nn4_W, nn4_b):
    bf = lambda t: t.astype(jnp.bfloat16)
    weights = []
    for (W1, b1, W2, b2) in ((c1_W1, c1_b1, c1_W2, c1_b2),
                             (c2_W1, c2_b1, c2_W2, c2_b2),
                             (c3_W1, c3_b1, c3_W2, c3_b2),
                             (c4_W1, c4_b1, c4_W2, c4_b2)):
        din = W1.shape[1] // 2
        weights += [bf(W1[:, :din].T), bf(W1[:, din:].T),
                    b1[None, :], bf(W2.T), b2[None, :]]

    s = bf(nn1_W.T)
    weights += [s[0:7], s[7:263], s[263:519], s[519:775], s[775:1031],
                nn1_b[None, :]]
    weights += [bf(nn2_W.T), nn2_b[None, :]]
    t = bf(nn3_W.T)
    weights += [t[0:256], t[256:512], t[512:768], t[768:1024], t[1024:1029],
                nn3_b[None, :]]
    weights += [bf(nn4_W.T), nn4_b[None, :]]

    np2 = n_pulses.reshape(B, 1, 1)

    wspecs = [pl.BlockSpec(w.shape, lambda g: (0, 0)) for w in weights]
    out = pl.pallas_call(
        _body,
        grid=(B,),
        in_specs=[pl.BlockSpec((NPER, 7), lambda g: (g, 0)),
                  pl.BlockSpec((1, 1, 1), lambda g: (g, 0, 0))] + wspecs,
        out_specs=pl.BlockSpec((1, 1, 3), lambda g: (g, 0, 0)),
        out_shape=jax.ShapeDtypeStruct((B, 1, 3), jnp.float32),
    )(x, np2, *weights)
    return out.reshape(B, 3)


# speedup vs baseline: 1.2171x; 1.0197x over previous
import jax
import jax.numpy as jnp
from jax.experimental import pallas as pl

N = 10000
B = 50
NPER = 200
K = 8
SLOPE = 0.01


def _bf(t):
    return t.astype(jnp.bfloat16)


def _mm(a, b):
    return jax.lax.dot_general(a, b, (((1,), (0,)), ((), ())),
                               preferred_element_type=jnp.float32)


def _leaky(t):
    return jnp.where(t >= 0, t, SLOPE * t)


def _split3(x):
    h1 = _bf(x)
    r = x - h1.astype(jnp.float32)
    h2 = _bf(r)
    h3 = _bf(r - h2.astype(jnp.float32))
    return h1, h2, h3


def _body(x_ref, np_ref,
          A1, B1, c1b1, W21, c1b2,
          A2, B2, c2b1, W22, c2b2,
          A3, B3, c3b1, W23, c3b2,
          A4, B4, c4b1, W24, c4b2,
          sx, sa, sb, sc, sd, nb1,
          n2T, nb2,
          tmx, tmn, tsm, tmean, textra, nb3,
          n4T, nb4,
          out_ref):
    Xall = x_ref[...]
    ci = jax.lax.broadcasted_iota(jnp.int32, (NPER, NPER), 1)
    ri = jax.lax.broadcasted_iota(jnp.int32, (NPER, NPER), 0)
    cif = ci.astype(jnp.float32)
    eyeb = (ri == ci).astype(jnp.bfloat16)
    eye_big = (ri == ci).astype(jnp.float32) * 1e10

    def knn(P):
        p1, p2, p3 = _split3(P)
        Pc = jnp.concatenate([p1, p2, p3], axis=1)
        PT = jax.lax.dot_general(Pc, eyeb, (((0,), (0,)), ((), ())),
                                 preferred_element_type=jnp.float32)
        d2 = None
        for c in range(3):
            col = P[:, c:c + 1]
            row = (PT[c:c + 1] + PT[c + 3:c + 4]) + PT[c + 6:c + 7]
            dc = col - row
            sq = dc * dc
            d2 = sq if d2 is None else d2 + sq
        d2 = d2 + eye_big
        ohs = []
        work = d2
        for _ in range(K):
            m = jnp.min(work, axis=1, keepdims=True)
            cand = jnp.where(work == m, cif, jnp.float32(1e9))
            amin = jnp.min(cand, axis=1, keepdims=True)
            oh = cif == amin
            ohs.append(oh)
            work = jnp.where(oh, jnp.float32(jnp.inf), work)
        return ohs

    def conv(Xin, cols, WaT, WbT, b1r, W2T, b2r, hom=False):
        chunks = _split3(Xin)
        Un = _mm(chunks[0], WaT[...])
        ohbs = [oh.astype(jnp.bfloat16) for oh in cols]
        g1s = [_mm(ohb, chunks[0]) for ohb in ohbs]
        g2s = [_mm(ohb, chunks[1]) for ohb in ohbs]
        g3s = [_mm(ohb, chunks[2]) for ohb in ohbs]
        Vks = [(g1 + g2) + g3 for g1, g2, g3 in zip(g1s, g2s, g3s)]
        Dks = [_bf(Vk - Xin) for Vk in Vks]
        P1s = [(Un + _mm(Dk, WbT[...])) + b1r[...] for Dk in Dks]
        E1s = [_bf(_leaky(P1)) for P1 in P1s]
        E2s = [_leaky(_mm(E1, W2T[...]) + b2r[...]) for E1 in E1s]
        acc = None
        for E2 in E2s:
            acc = E2 if acc is None else acc + E2
        if not hom:
            return acc
        hsum = None
        for Vk in Vks:
            same = (Vk[:, 0:4] == Xin[:, 0:4]).astype(jnp.float32)
            s = jnp.sum(same, axis=0, keepdims=True)
            hsum = s if hsum is None else hsum + s
        return acc, hsum * (1.0 / (NPER * K))

    def one_graph(X, npul):
        src1 = knn(X[:, 0:3])
        a, h4 = conv(X, src1, A1, B1, c1b1, W21, c1b2, hom=True)
        src2 = knn(a[:, 0:3])
        b = conv(a, src2, A2, B2, c2b1, W22, c2b2)
        src3 = knn(b[:, 0:3])
        c = conv(b, src3, A3, B3, c3b1, W23, c3b2)
        src4 = knn(c[:, 0:3])
        d = conv(c, src4, A4, B4, c4b1, W24, c4b2)

        H = _leaky((((( _mm(_bf(X), sx[...]) + _mm(_bf(a), sa[...]))
                     + _mm(_bf(b), sb[...])) + _mm(_bf(c), sc[...]))
                    + _mm(_bf(d), sd[...])) + nb1[...])
        H2 = _mm(_bf(H), n2T[...]) + nb2[...]

        mx = jnp.max(H2, axis=0, keepdims=True)
        mn = jnp.min(H2, axis=0, keepdims=True)
        sm = jnp.sum(H2, axis=0, keepdims=True)
        mean = sm * (1.0 / NPER)

        ev = jnp.concatenate([h4[:, 3:4], h4[:, 0:1], h4[:, 1:2],
                              h4[:, 2:3], npul], axis=1)

        G = ((((_mm(_bf(_leaky(mx)), tmx[...])
                + _mm(_bf(_leaky(mn)), tmn[...]))
               + _mm(_bf(_leaky(sm)), tsm[...]))
              + _mm(_bf(_leaky(mean)), tmean[...]))
             + _mm(_bf(_leaky(ev)), textra[...]))
        G = _leaky(G + nb3[...])
        O = _mm(_bf(G), n4T[...]) + nb4[...]
        return jnp.concatenate([jnp.tanh(O[:, 0:2]), O[:, 2:3]], axis=1)

    o0 = one_graph(Xall[0:NPER], np_ref[0, 0:1])
    o1 = one_graph(Xall[NPER:2 * NPER], np_ref[0, 1:2])
    out_ref[0] = jnp.concatenate([o0, o1], axis=0)


def kernel(x, batch, n_pulses,
           c1_W1, c1_b1, c1_W2, c1_b2,
           c2_W1, c2_b1, c2_W2, c2_b2,
           c3_W1, c3_b1, c3_W2, c3_b2,
           c4_W1, c4_b1, c4_W2, c4_b2,
           nn1_W, nn1_b, nn2_W, nn2_b,
           nn3_W, nn3_b, nn4_W, nn4_b):
    bf = lambda t: t.astype(jnp.bfloat16)
    weights = []
    for (W1, b1, W2, b2) in ((c1_W1, c1_b1, c1_W2, c1_b2),
                             (c2_W1, c2_b1, c2_W2, c2_b2),
                             (c3_W1, c3_b1, c3_W2, c3_b2),
                             (c4_W1, c4_b1, c4_W2, c4_b2)):
        din = W1.shape[1] // 2
        weights += [bf(W1[:, :din].T), bf(W1[:, din:].T),
                    b1[None, :], bf(W2.T), b2[None, :]]

    s = bf(nn1_W.T)
    weights += [s[0:7], s[7:263], s[263:519], s[519:775], s[775:1031],
                nn1_b[None, :]]
    weights += [bf(nn2_W.T), nn2_b[None, :]]
    t = bf(nn3_W.T)
    weights += [t[0:256], t[256:512], t[512:768], t[768:1024], t[1024:1029],
                nn3_b[None, :]]
    weights += [bf(nn4_W.T), nn4_b[None, :]]

    np2 = n_pulses.reshape(B // 2, 2, 1)

    wspecs = [pl.BlockSpec(w.shape, lambda g: (0, 0)) for w in weights]
    out = pl.pallas_call(
        _body,
        grid=(B // 2,),
        in_specs=[pl.BlockSpec((2 * NPER, 7), lambda g: (g, 0)),
                  pl.BlockSpec((1, 2, 1), lambda g: (g, 0, 0))] + wspecs,
        out_specs=pl.BlockSpec((1, 2, 3), lambda g: (g, 0, 0)),
        out_shape=jax.ShapeDtypeStruct((B // 2, 2, 3), jnp.float32),
    )(x, np2, *weights)
    return out.reshape(B, 3)
